# baseline (device time: 212553 ns/iter reference)
import jax
import jax.numpy as jnp
from jax import lax
from jax.experimental import pallas as pl
from jax.experimental.pallas import tpu as pltpu

N_DEV = 16
M_BLK = 256


def kernel(x, w_mat):
    x = x.astype(jnp.bfloat16)
    w = w_mat.astype(jnp.bfloat16)
    m, k = x.shape
    _, n = w.shape

    def body(x_ref, w_ref, out_ref, send_buf, recv_buf, send_sems, recv_sems):
        my = lax.axis_index("i")
        left = lax.rem(my - 1 + N_DEV, N_DEV)
        right = lax.rem(my + 1, N_DEV)

        barrier = pltpu.get_barrier_semaphore()
        for nbr in (left, right):
            pl.semaphore_signal(
                barrier, inc=1, device_id=(nbr,),
                device_id_type=pl.DeviceIdType.MESH,
            )
        pl.semaphore_wait(barrier, 2)

        def chunk_gemm(c):
            a = x_ref[pl.ds(c * M_BLK, M_BLK), :]
            return jnp.dot(a, w_ref[...], preferred_element_type=jnp.float32)

        for t in range(N_DEV - 1):
            c = lax.rem(my - 1 - t + 2 * N_DEV, N_DEV)
            acc = chunk_gemm(c)
            if t > 0:
                acc = acc + recv_buf[t - 1].astype(jnp.float32)
            send_buf[t % 2] = acc.astype(jnp.bfloat16)
            rdma = pltpu.make_async_remote_copy(
                src_ref=send_buf.at[t % 2],
                dst_ref=recv_buf.at[t],
                send_sem=send_sems.at[t % 2],
                recv_sem=recv_sems.at[t],
                device_id=(right,),
                device_id_type=pl.DeviceIdType.MESH,
            )
            rdma.start()
            rdma.wait()

        own = chunk_gemm(my) + recv_buf[N_DEV - 2].astype(jnp.float32)
        out_ref[...] = jnp.maximum(own, 0.0)

    return pl.pallas_call(
        body,
        out_shape=jax.ShapeDtypeStruct((M_BLK, n), jnp.float32),
        in_specs=[
            pl.BlockSpec(memory_space=pltpu.VMEM),
            pl.BlockSpec(memory_space=pltpu.VMEM),
        ],
        out_specs=pl.BlockSpec(memory_space=pltpu.VMEM),
        scratch_shapes=[
            pltpu.VMEM((2, M_BLK, n), jnp.bfloat16),
            pltpu.VMEM((N_DEV - 1, M_BLK, n), jnp.bfloat16),
            pltpu.SemaphoreType.DMA((2,)),
            pltpu.SemaphoreType.DMA((N_DEV - 1,)),
        ],
        compiler_params=pltpu.CompilerParams(collective_id=0),
    )(x, w)


# device time: 150336 ns/iter; 1.4139x vs baseline; 1.4139x over previous
import jax
import jax.numpy as jnp
from jax import lax
from jax.experimental import pallas as pl
from jax.experimental.pallas import tpu as pltpu

N_DEV = 16
M_BLK = 256


def kernel(x, w_mat):
    x = x.astype(jnp.bfloat16)
    w = w_mat.astype(jnp.bfloat16)
    m, k = x.shape
    _, n = w.shape
    nh = n // 2

    def body(x_ref, w_ref, out_ref,
             send_a, send_b, recv_a, recv_b,
             send_a_sems, send_b_sems, recv_a_sems, recv_b_sems):
        my = lax.axis_index("i")
        left = lax.rem(my - 1 + N_DEV, N_DEV)
        right = lax.rem(my + 1, N_DEV)

        barrier = pltpu.get_barrier_semaphore()
        for nbr in (left, right):
            pl.semaphore_signal(
                barrier, inc=1, device_id=(nbr,),
                device_id_type=pl.DeviceIdType.MESH,
            )
        pl.semaphore_wait(barrier, 2)

        def gemm_a(c):
            a = x_ref[pl.ds(c * M_BLK, M_BLK), :]
            return jnp.dot(a, w_ref[:, :nh], preferred_element_type=jnp.float32)

        def gemm_b(c):
            a = x_ref[pl.ds(c * M_BLK, M_BLK), :]
            return jnp.dot(a, w_ref[:, nh:], preferred_element_type=jnp.float32)

        rdmas_a = []
        rdmas_b = []
        for t in range(N_DEV - 1):
            ca = lax.rem(my - 1 - t + 2 * N_DEV, N_DEV)
            cb = lax.rem(my + 1 + t, N_DEV)

            acc_a = gemm_a(ca)
            acc_b = gemm_b(cb)
            if t > 0:
                rdmas_a[t - 1].wait_recv()
                rdmas_b[t - 1].wait_recv()
                acc_a = acc_a + recv_a[t - 1].astype(jnp.float32)
                acc_b = acc_b + recv_b[t - 1].astype(jnp.float32)
            if t >= 2:
                rdmas_a[t - 2].wait_send()
                rdmas_b[t - 2].wait_send()
            send_a[t % 2] = acc_a.astype(jnp.bfloat16)
            send_b[t % 2] = acc_b.astype(jnp.bfloat16)

            rdma_a = pltpu.make_async_remote_copy(
                src_ref=send_a.at[t % 2],
                dst_ref=recv_a.at[t],
                send_sem=send_a_sems.at[t % 2],
                recv_sem=recv_a_sems.at[t],
                device_id=(right,),
                device_id_type=pl.DeviceIdType.MESH,
            )
            rdma_b = pltpu.make_async_remote_copy(
                src_ref=send_b.at[t % 2],
                dst_ref=recv_b.at[t],
                send_sem=send_b_sems.at[t % 2],
                recv_sem=recv_b_sems.at[t],
                device_id=(left,),
                device_id_type=pl.DeviceIdType.MESH,
            )
            rdma_a.start()
            rdma_b.start()
            rdmas_a.append(rdma_a)
            rdmas_b.append(rdma_b)

        own_a = gemm_a(my)
        own_b = gemm_b(my)
        rdmas_a[N_DEV - 2].wait_recv()
        rdmas_b[N_DEV - 2].wait_recv()
        out_ref[:, :nh] = jnp.maximum(
            own_a + recv_a[N_DEV - 2].astype(jnp.float32), 0.0)
        out_ref[:, nh:] = jnp.maximum(
            own_b + recv_b[N_DEV - 2].astype(jnp.float32), 0.0)

        for t in (N_DEV - 3, N_DEV - 2):
            rdmas_a[t].wait_send()
            rdmas_b[t].wait_send()

    return pl.pallas_call(
        body,
        out_shape=jax.ShapeDtypeStruct((M_BLK, n), jnp.float32),
        in_specs=[
            pl.BlockSpec(memory_space=pltpu.VMEM),
            pl.BlockSpec(memory_space=pltpu.VMEM),
        ],
        out_specs=pl.BlockSpec(memory_space=pltpu.VMEM),
        scratch_shapes=[
            pltpu.VMEM((2, M_BLK, nh), jnp.bfloat16),
            pltpu.VMEM((2, M_BLK, nh), jnp.bfloat16),
            pltpu.VMEM((N_DEV - 1, M_BLK, nh), jnp.bfloat16),
            pltpu.VMEM((N_DEV - 1, M_BLK, nh), jnp.bfloat16),
            pltpu.SemaphoreType.DMA((2,)),
            pltpu.SemaphoreType.DMA((2,)),
            pltpu.SemaphoreType.DMA((N_DEV - 1,)),
            pltpu.SemaphoreType.DMA((N_DEV - 1,)),
        ],
        compiler_params=pltpu.CompilerParams(collective_id=0),
    )(x, w)


# device time: 129149 ns/iter; 1.6458x vs baseline; 1.1641x over previous
import jax
import jax.numpy as jnp
from jax import lax
from jax.experimental import pallas as pl
from jax.experimental.pallas import tpu as pltpu

N_DEV = 16
M_BLK = 256

RING = (0, 1, 5, 9, 13, 14, 10, 6, 2, 3, 7, 11, 15, 12, 8, 4)
POS = tuple(RING.index(l) for l in range(N_DEV))


def _lut(table, idx, jnp):
    out = jnp.int32(table[0])
    for q in range(1, N_DEV):
        out = jnp.where(idx == q, jnp.int32(table[q]), out)
    return out


def kernel(x, w_mat):
    x = x.astype(jnp.bfloat16)
    w = w_mat.astype(jnp.bfloat16)
    m, k = x.shape
    _, n = w.shape
    nh = n // 2

    def body(x_ref, w_ref, out_ref,
             send_a, send_b, recv_a, recv_b,
             send_a_sems, send_b_sems, recv_a_sems, recv_b_sems):
        my = lax.axis_index("i")
        pos = _lut(POS, my, jnp)
        right = _lut(tuple(RING[(POS[l] + 1) % N_DEV] for l in range(N_DEV)),
                     my, jnp)
        left = _lut(tuple(RING[(POS[l] - 1) % N_DEV] for l in range(N_DEV)),
                    my, jnp)

        barrier = pltpu.get_barrier_semaphore()
        for nbr in (left, right):
            pl.semaphore_signal(
                barrier, inc=1, device_id=(nbr,),
                device_id_type=pl.DeviceIdType.MESH,
            )
        pl.semaphore_wait(barrier, 2)

        def gemm_a(c):
            a = x_ref[pl.ds(c * M_BLK, M_BLK), :]
            return jnp.dot(a, w_ref[:, :nh], preferred_element_type=jnp.float32)

        def gemm_b(c):
            a = x_ref[pl.ds(c * M_BLK, M_BLK), :]
            return jnp.dot(a, w_ref[:, nh:], preferred_element_type=jnp.float32)

        rdmas_a = []
        rdmas_b = []
        for t in range(N_DEV - 1):
            ca = _lut(tuple(RING[(q - 1 - t) % N_DEV] for q in range(N_DEV)),
                      pos, jnp)
            cb = _lut(tuple(RING[(q + 1 + t) % N_DEV] for q in range(N_DEV)),
                      pos, jnp)

            acc_a = gemm_a(ca)
            acc_b = gemm_b(cb)
            if t > 0:
                rdmas_a[t - 1].wait_recv()
                rdmas_b[t - 1].wait_recv()
                acc_a = acc_a + recv_a[t - 1].astype(jnp.float32)
                acc_b = acc_b + recv_b[t - 1].astype(jnp.float32)
            if t >= 2:
                rdmas_a[t - 2].wait_send()
                rdmas_b[t - 2].wait_send()
            send_a[t % 2] = acc_a.astype(jnp.bfloat16)
            send_b[t % 2] = acc_b.astype(jnp.bfloat16)

            rdma_a = pltpu.make_async_remote_copy(
                src_ref=send_a.at[t % 2],
                dst_ref=recv_a.at[t],
                send_sem=send_a_sems.at[t % 2],
                recv_sem=recv_a_sems.at[t],
                device_id=(right,),
                device_id_type=pl.DeviceIdType.MESH,
            )
            rdma_b = pltpu.make_async_remote_copy(
                src_ref=send_b.at[t % 2],
                dst_ref=recv_b.at[t],
                send_sem=send_b_sems.at[t % 2],
                recv_sem=recv_b_sems.at[t],
                device_id=(left,),
                device_id_type=pl.DeviceIdType.MESH,
            )
            rdma_a.start()
            rdma_b.start()
            rdmas_a.append(rdma_a)
            rdmas_b.append(rdma_b)

        own_a = gemm_a(my)
        own_b = gemm_b(my)
        rdmas_a[N_DEV - 2].wait_recv()
        rdmas_b[N_DEV - 2].wait_recv()
        out_ref[:, :nh] = jnp.maximum(
            own_a + recv_a[N_DEV - 2].astype(jnp.float32), 0.0)
        out_ref[:, nh:] = jnp.maximum(
            own_b + recv_b[N_DEV - 2].astype(jnp.float32), 0.0)

        for t in (N_DEV - 3, N_DEV - 2):
            rdmas_a[t].wait_send()
            rdmas_b[t].wait_send()

    return pl.pallas_call(
        body,
        out_shape=jax.ShapeDtypeStruct((M_BLK, n), jnp.float32),
        in_specs=[
            pl.BlockSpec(memory_space=pltpu.VMEM),
            pl.BlockSpec(memory_space=pltpu.VMEM),
        ],
        out_specs=pl.BlockSpec(memory_space=pltpu.VMEM),
        scratch_shapes=[
            pltpu.VMEM((2, M_BLK, nh), jnp.bfloat16),
            pltpu.VMEM((2, M_BLK, nh), jnp.bfloat16),
            pltpu.VMEM((N_DEV - 1, M_BLK, nh), jnp.bfloat16),
            pltpu.VMEM((N_DEV - 1, M_BLK, nh), jnp.bfloat16),
            pltpu.SemaphoreType.DMA((2,)),
            pltpu.SemaphoreType.DMA((2,)),
            pltpu.SemaphoreType.DMA((N_DEV - 1,)),
            pltpu.SemaphoreType.DMA((N_DEV - 1,)),
        ],
        compiler_params=pltpu.CompilerParams(collective_id=0),
    )(x, w)


# device time: 95975 ns/iter; 2.2147x vs baseline; 1.3457x over previous
import jax
import jax.numpy as jnp
from jax import lax
from jax.experimental import pallas as pl
from jax.experimental.pallas import tpu as pltpu

N_DEV = 16
M_BLK = 256

RING = (0, 1, 5, 9, 13, 14, 10, 6, 2, 3, 7, 11, 15, 12, 8, 4)
POS = tuple(RING.index(l) for l in range(N_DEV))


def _lut(table, idx):
    out = jnp.int32(table[0])
    for q in range(1, N_DEV):
        out = jnp.where(idx == q, jnp.int32(table[q]), out)
    return out


class _Lane:

    def __init__(self, col0, col1, send, recv, send_sems, recv_sems):
        self.col0 = col0
        self.col1 = col1
        self.send = send
        self.recv = recv
        self.send_sems = send_sems
        self.recv_sems = recv_sems
        self.rdmas = []


def kernel(x, w_mat):
    x = x.astype(jnp.bfloat16)
    w = w_mat.astype(jnp.bfloat16)
    m, k = x.shape
    _, n = w.shape
    nq = n // 4

    def body(x_ref, w_ref, out_ref,
             send_a0, send_a1, send_b0, send_b1,
             recv_a0, recv_a1, recv_b0, recv_b1,
             ss_a0, ss_a1, ss_b0, ss_b1,
             rs_a0, rs_a1, rs_b0, rs_b1):
        my = lax.axis_index("i")
        pos = _lut(POS, my)
        right = _lut(tuple(RING[(POS[l] + 1) % N_DEV] for l in range(N_DEV)),
                     my)
        left = _lut(tuple(RING[(POS[l] - 1) % N_DEV] for l in range(N_DEV)),
                    my)

        barrier = pltpu.get_barrier_semaphore()
        for nbr in (left, right):
            pl.semaphore_signal(
                barrier, inc=1, device_id=(nbr,),
                device_id_type=pl.DeviceIdType.MESH,
            )
        pl.semaphore_wait(barrier, 2)

        lanes = [
            _Lane(0 * nq, 1 * nq, send_a0, recv_a0, ss_a0, rs_a0),
            _Lane(2 * nq, 3 * nq, send_b0, recv_b0, ss_b0, rs_b0),
            _Lane(1 * nq, 2 * nq, send_a1, recv_a1, ss_a1, rs_a1),
            _Lane(3 * nq, 4 * nq, send_b1, recv_b1, ss_b1, rs_b1),
        ]
        lane_nbr = [right, left, right, left]
        lane_is_a = [True, False, True, False]

        def gemm(c, lane):
            a = x_ref[pl.ds(c * M_BLK, M_BLK), :]
            return jnp.dot(a, w_ref[:, lane.col0:lane.col1],
                           preferred_element_type=jnp.float32)

        for t in range(N_DEV - 1):
            ca = _lut(tuple(RING[(q - 1 - t) % N_DEV] for q in range(N_DEV)),
                      pos)
            cb = _lut(tuple(RING[(q + 1 + t) % N_DEV] for q in range(N_DEV)),
                      pos)
            for lane, nbr, is_a in zip(lanes, lane_nbr, lane_is_a):
                acc = gemm(ca if is_a else cb, lane)
                if t > 0:
                    lane.rdmas[t - 1].wait_recv()
                    acc = acc + lane.recv[t - 1].astype(jnp.float32)
                if t >= 2:
                    lane.rdmas[t - 2].wait_send()
                lane.send[t % 2] = acc.astype(jnp.bfloat16)
                rdma = pltpu.make_async_remote_copy(
                    src_ref=lane.send.at[t % 2],
                    dst_ref=lane.recv.at[t],
                    send_sem=lane.send_sems.at[t % 2],
                    recv_sem=lane.recv_sems.at[t],
                    device_id=(nbr,),
                    device_id_type=pl.DeviceIdType.MESH,
                )
                rdma.start()
                lane.rdmas.append(rdma)

        for lane in lanes:
            own = gemm(my, lane)
            lane.rdmas[N_DEV - 2].wait_recv()
            out_ref[:, lane.col0:lane.col1] = jnp.maximum(
                own + lane.recv[N_DEV - 2].astype(jnp.float32), 0.0)

        for lane in lanes:
            lane.rdmas[N_DEV - 3].wait_send()
            lane.rdmas[N_DEV - 2].wait_send()

    return pl.pallas_call(
        body,
        out_shape=jax.ShapeDtypeStruct((M_BLK, n), jnp.float32),
        in_specs=[
            pl.BlockSpec(memory_space=pltpu.VMEM),
            pl.BlockSpec(memory_space=pltpu.VMEM),
        ],
        out_specs=pl.BlockSpec(memory_space=pltpu.VMEM),
        scratch_shapes=(
            [pltpu.VMEM((2, M_BLK, nq), jnp.bfloat16)] * 4
            + [pltpu.VMEM((N_DEV - 1, M_BLK, nq), jnp.bfloat16)] * 4
            + [pltpu.SemaphoreType.DMA((2,))] * 4
            + [pltpu.SemaphoreType.DMA((N_DEV - 1,))] * 4
        ),
        compiler_params=pltpu.CompilerParams(collective_id=0),
    )(x, w)
